# Initial kernel scaffold; baseline (speedup 1.0000x reference)
import jax
import jax.numpy as jnp
from jax.experimental import pallas as pl


def kernel(x, perb, edge_index, W_online, b_online, W_target, b_target, W1, b1, gamma, beta, prelu_a, W2, b2):
    raise NotImplementedError("write your pallas kernel here")


# speedup vs baseline: 5.8529x; 5.8529x over previous
import functools

import jax
import jax.numpy as jnp
from jax import lax
from jax.experimental import pallas as pl
from jax.experimental.pallas import tpu as pltpu
from jax.experimental.pallas import tpu_sc as plsc

N = 10000
E = 320000
D = 128
BN_EPS = 1e-5

NC = 2
NS = 16
EPC = E // NS
CH = 80
NCHUNK = EPC // CH
NP = 10240
RPS = NP // NS
ZR = 80

@functools.cache
def _get_sc_segsum():
    mesh = plsc.VectorSubcoreMesh(
        core_axis_name="c", subcore_axis_name="s",
        num_cores=NC, num_subcores=NS)
    return functools.partial(
        pl.kernel,
        out_type=jax.ShapeDtypeStruct((NC, NP, D), jnp.float32),
        mesh=mesh,
        scratch_types=[
            pltpu.VMEM((CH,), jnp.int32),
            pltpu.VMEM((CH,), jnp.int32),
            pltpu.VMEM((CH, D), jnp.float32),
            pltpu.VMEM((ZR, D), jnp.float32),
            pltpu.VMEM_SHARED((NP, D), jnp.float32),
            pltpu.SemaphoreType.DMA,
        ],
    )(_sc_segsum_body)


def _sc_segsum_body(src_hbm, col2_hbm, row_hbm, out_hbm,
                    colv, rowv, gbuf, zbuf, acc, sem):
    c = lax.axis_index("c")
    s = lax.axis_index("s")

    for r in range(ZR):
        for j in range(D // 16):
            zbuf[r, 16 * j:16 * (j + 1)] = jnp.zeros((16,), jnp.float32)
    for k in range(RPS // ZR):
        pltpu.sync_copy(zbuf, acc.at[pl.ds(s * RPS + k * ZR, ZR)])
    plsc.subcore_barrier()

    def body(i, carry):
        base = s * EPC + i * CH
        pltpu.sync_copy(col2_hbm.at[pl.ds(c * E + base, CH)], colv)
        pltpu.sync_copy(row_hbm.at[pl.ds(base, CH)], rowv)
        pltpu.async_copy(src_hbm.at[colv], gbuf, sem).wait()
        pltpu.sync_copy(gbuf, acc.at[rowv], add=True)
        return carry

    lax.fori_loop(0, NCHUNK, body, 0)
    plsc.subcore_barrier()

    pltpu.sync_copy(acc.at[pl.ds(s * RPS, RPS)],
                    out_hbm.at[c, pl.ds(s * RPS, RPS)])


def _tc_body(x_ref, perb_ref, s0_ref, s1_ref,
             wo_ref, bo_ref, wt_ref, bt_ref,
             w1t_ref, b1_ref, gamma_ref, beta_ref, a_ref, w2t_ref, b2_ref,
             embed_ref, loss_ref):
    s0 = s0_ref[...]
    s01 = s0 + s1_ref[...]

    wo = wo_ref[...]
    bo = bo_ref[...]
    g1 = jnp.dot(s0, wo, preferred_element_type=jnp.float32) + bo
    g2 = jnp.dot(s01, wo, preferred_element_type=jnp.float32) + bo
    embed_ref[...] = x_ref[...] + perb_ref[...] + g2

    wt = wt_ref[...]
    bt = bt_ref[...]
    t1 = jnp.dot(s0, wt, preferred_element_type=jnp.float32) + bt
    t2 = jnp.dot(s01, wt, preferred_element_type=jnp.float32) + bt

    w1t = w1t_ref[...]
    w2t = w2t_ref[...]
    b1 = b1_ref[...]
    b2 = b2_ref[...]
    gamma = gamma_ref[...]
    beta = beta_ref[...]
    a = a_ref[0, 0]

    def predictor(z):
        h = jnp.dot(z, w1t, preferred_element_type=jnp.float32) + b1
        mean = jnp.mean(h, axis=0, keepdims=True)
        d = h - mean
        var = jnp.mean(d * d, axis=0, keepdims=True)
        h = gamma * d * jax.lax.rsqrt(var + BN_EPS) + beta
        h = jnp.where(h >= 0.0, h, a * h)
        return jnp.dot(h, w2t, preferred_element_type=jnp.float32) + b2

    p1 = predictor(g1)
    p2 = predictor(g2)

    def l2n(v):
        ss = jnp.sum(v * v, axis=-1, keepdims=True)
        return v / jnp.maximum(jnp.sqrt(ss), 1e-12)

    dots = jnp.sum(l2n(p1) * l2n(t2) + l2n(p2) * l2n(t1))
    loss_ref[0, 0] = 4.0 - 2.0 * dots / N


def _tc_stage(x, perb, s0, s1, W_online, b_online, W_target, b_target,
              W1, b1, gamma, beta, prelu_a, W2, b2):
    vmem = pl.BlockSpec(memory_space=pltpu.VMEM)
    smem = pl.BlockSpec(memory_space=pltpu.SMEM)
    embed, loss = pl.pallas_call(
        _tc_body,
        out_shape=[
            jax.ShapeDtypeStruct((N, D), jnp.float32),
            jax.ShapeDtypeStruct((1, 1), jnp.float32),
        ],
        in_specs=[vmem] * 12 + [smem] + [vmem] * 2,
        out_specs=[vmem, smem],
    )(x, perb, s0, s1,
      W_online, b_online.reshape(1, D), W_target, b_target.reshape(1, D),
      W1.T, b1.reshape(1, D), gamma.reshape(1, D), beta.reshape(1, D),
      prelu_a.reshape(1, 1), W2.T, b2.reshape(1, D))
    return embed, loss[0, 0]


def kernel(x, perb, edge_index, W_online, b_online, W_target, b_target,
           W1, b1, gamma, beta, prelu_a, W2, b2):
    row = edge_index[0]
    col = edge_index[1]
    src = jnp.concatenate([x, perb], axis=0)
    col2 = jnp.concatenate([col, col + N], axis=0)
    agg = _get_sc_segsum()(src, col2, row)[:, :N, :]
    embed, loss = _tc_stage(x, perb, agg[0], agg[1],
                            W_online, b_online, W_target, b_target,
                            W1, b1, gamma, beta, prelu_a, W2, b2)
    return (embed, loss)
